# baseline (speedup 1.0000x reference)
import functools

import jax
import jax.numpy as jnp
from jax import lax
from jax.experimental import pallas as pl
from jax.experimental.pallas import tpu as pltpu
from jax.experimental.pallas import tpu_sc as plsc

NUM_SEGMENTS = 256
D = 128
LANES = 16
NVR = D // LANES
CHUNK = 256
ACC_ROWS = NUM_SEGMENTS + LANES


def _sc_weighted_segment_sum(x, batch, wb, *, row0, n_rows, n_chunks):
    mesh = plsc.VectorSubcoreMesh(core_axis_name="c", subcore_axis_name="s")
    info = plsc.get_sparse_core_info()
    nc, ns = info.num_cores, info.num_subcores
    nw = nc * ns
    oct0 = row0 // 8
    n_oct = (n_rows - row0) // 8

    @functools.partial(
        pl.kernel,
        mesh=mesh,
        out_type=jax.ShapeDtypeStruct((nc, NUM_SEGMENTS, D), jnp.float32),
        scratch_types=[
            pltpu.VMEM((CHUNK, D), jnp.float32),
            pltpu.VMEM((CHUNK, D), jnp.float32),
            pltpu.VMEM((CHUNK + 8,), jnp.int32),
            pltpu.VMEM((CHUNK + 8,), jnp.int32),
            pltpu.VMEM((D + LANES,), jnp.float32),
            pltpu.VMEM((ACC_ROWS, D), jnp.float32),
            pltpu.VMEM((ns, D), jnp.float32),
            pltpu.VMEM((LANES,), jnp.int32),
            pltpu.VMEM_SHARED((NUM_SEGMENTS + 1, D), jnp.float32),
            pltpu.SemaphoreType.DMA,
            pltpu.SemaphoreType.DMA,
            pltpu.SemaphoreType.DMA,
            pltpu.SemaphoreType.DMA,
        ],
    )
    def run(x_hbm, b_hbm, wb_hbm, out_hbm,
            xb0, xb1, bb0, bb1, wv, accum, zbuf, sbi, shared,
            sx0, sx1, sb0, sb1):
        cid = lax.axis_index("c")
        sid = lax.axis_index("s")
        wid = sid * nc + cid

        xbufs = (xb0, xb1)
        bbufs = (bb0, bb1)
        sxs = (sx0, sx1)
        sbs = (sb0, sb1)

        o0 = oct0 + wid * n_oct // nw
        o1 = oct0 + (wid + 1) * n_oct // nw
        r0 = o0 * 8
        r1 = o1 * 8

        pltpu.sync_copy(wb_hbm, wv)

        zvec = jnp.zeros((LANES,), jnp.float32)

        def zacc_body(r, _):
            for d in range(NVR):
                accum[r, pl.ds(d * LANES, LANES)] = zvec
            return 0
        lax.fori_loop(0, ACC_ROWS, zacc_body, 0)

        for r in range(ns):
            for d in range(NVR):
                zbuf[r, pl.ds(d * LANES, LANES)] = zvec
        pltpu.sync_copy(zbuf, shared.at[pl.ds(sid * ns, ns)])
        plsc.subcore_barrier()

        def chunk_base(k):
            return jnp.minimum(r0 + k * CHUNK, r1 - CHUNK)

        def start_copy(k, parity):
            base = chunk_base(k)
            pltpu.make_async_copy(
                x_hbm.at[pl.ds(base, CHUNK)], xbufs[parity], sxs[parity]
            ).start()
            pltpu.make_async_copy(
                b_hbm.at[pl.ds(base, CHUNK)],
                bbufs[parity].at[pl.ds(0, CHUNK)],
                sbs[parity],
            ).start()

        def wait_copy(k, parity):
            base = chunk_base(k)
            pltpu.make_async_copy(
                x_hbm.at[pl.ds(base, CHUNK)], xbufs[parity], sxs[parity]
            ).wait()
            pltpu.make_async_copy(
                b_hbm.at[pl.ds(base, CHUNK)],
                bbufs[parity].at[pl.ds(0, CHUNK)],
                sbs[parity],
            ).wait()

        wregs = [wv[pl.ds(d * LANES, LANES)] for d in range(NVR)]
        bias = wv[pl.ds(D, LANES)]

        lane_iota = lax.iota(jnp.int32, LANES)
        perms = [lane_iota ^ sh for sh in (8, 4, 2, 1)]

        def lane_sum(m):
            for p in perms:
                m = m + m.at[p].get(mode="promise_in_bounds")
            return m

        def flush(cur_seg, acc):
            @pl.when(cur_seg >= 0)
            def _():
                for d in range(NVR):
                    sl = pl.ds(d * LANES, LANES)
                    accum[cur_seg, sl] = accum[cur_seg, sl] + acc[d]

        def process_chunk(carry, parity, base, fresh, tail):
            xb = xbufs[parity]
            bb = bbufs[parity]

            def grp_body(g, c):
                segv = bb[pl.ds(g * 16, LANES)]
                for i in range(16):
                    cur_seg = c[0]
                    acc = c[1:]
                    j = g * 16 + i
                    xv = [xb[j, pl.ds(d * LANES, LANES)] for d in range(NVR)]
                    m = xv[0] * wregs[0]
                    for d in range(1, NVR):
                        m = m + xv[d] * wregs[d]
                    z = lane_sum(m) + bias
                    w = 1.0 / (1.0 + jnp.exp(-z))
                    if tail:
                        valid = ((base + j) >= fresh).astype(jnp.float32)
                        w = w * valid
                    contrib = [w * xv[d] for d in range(NVR)]
                    seg = segv[i]
                    is_new = seg != cur_seg

                    @pl.when(jnp.logical_and(is_new, cur_seg >= 0))
                    def _(cur_seg=cur_seg, acc=acc):
                        for d in range(NVR):
                            sl = pl.ds(d * LANES, LANES)
                            accum[cur_seg, sl] = accum[cur_seg, sl] + acc[d]

                    c = (seg,) + tuple(
                        jnp.where(is_new, cb, a + cb)
                        for a, cb in zip(acc, contrib)
                    )
                return c

            return lax.fori_loop(0, CHUNK // 16, grp_body, carry)

        pltpu.sync_copy(b_hbm.at[pl.ds(r0, LANES)], sbi)
        seg_lo = sbi[pl.ds(0, LANES)][0]

        start_copy(0, 0)
        carry = (jnp.int32(-1),) + tuple(
            jnp.zeros((LANES,), jnp.float32) for _ in range(NVR)
        )

        n_pairs = (n_chunks - 1) // 2

        def pair_body(k2, c):
            for p in range(2):
                k = 2 * k2 + p
                wait_copy(k, p)
                start_copy(k + 1, (p + 1) % 2)
                c = process_chunk(c, p, chunk_base(k), r0 + k * CHUNK,
                                  tail=False)
            return c

        carry = lax.fori_loop(0, n_pairs, pair_body, carry)
        for k in range(2 * n_pairs, n_chunks):
            wait_copy(k, k % 2)
            if k + 1 < n_chunks:
                start_copy(k + 1, (k + 1) % 2)
            carry = process_chunk(carry, k % 2, chunk_base(k),
                                  r0 + k * CHUNK, tail=k == n_chunks - 1)

        seg_hi = carry[0]
        flush(seg_hi, carry[1:])

        iota = lax.iota(jnp.int32, LANES)
        for w in range(ns):
            start = seg_lo + w * LANES

            @pl.when(start <= seg_hi)
            def _(start=start):
                idxvec = jnp.minimum(start + iota, NUM_SEGMENTS)
                pltpu.sync_copy(
                    accum.at[pl.ds(start, LANES)],
                    shared.at[idxvec],
                    add=True,
                )

        plsc.subcore_barrier()

        pltpu.sync_copy(shared.at[pl.ds(sid * ns, ns)], zbuf)
        pltpu.sync_copy(zbuf, out_hbm.at[cid].at[pl.ds(sid * ns, ns)])

    return run(x, batch, wb)


TC_BLK = 1024
TC_SPLIT = 61440


def _tc_body(bb_ref, io_ref, x_ref, w_ref, b_ref, o_ref):
    i = pl.program_id(0)
    xb = x_ref[...]
    z = jnp.dot(xb, w_ref[...], preferred_element_type=jnp.float32)
    z = z + b_ref[...]
    wgt = 1.0 / (1.0 + jnp.exp(-z))
    y = (wgt * xb).astype(jnp.bfloat16)
    segb = bb_ref[0, 0, :].astype(jnp.bfloat16)
    onehot = (io_ref[...] == segb[None, :]).astype(jnp.bfloat16)
    part = jnp.dot(onehot, y, preferred_element_type=jnp.float32)

    @pl.when(i == 0)
    def _():
        o_ref[...] = jnp.zeros_like(o_ref)

    o_ref[...] += part


def _tc_weighted_segment_sum(x, batch, W, b):
    grid = TC_SPLIT // TC_BLK
    bb = batch[:TC_SPLIT].astype(jnp.int32).reshape(grid, 1, TC_BLK)
    iota_b = lax.broadcasted_iota(
        jnp.int32, (NUM_SEGMENTS, TC_BLK), 0
    ).astype(jnp.bfloat16)
    return pl.pallas_call(
        _tc_body,
        grid=(grid,),
        in_specs=[
            pl.BlockSpec((1, 1, TC_BLK), lambda i: (i, 0, 0)),
            pl.BlockSpec((NUM_SEGMENTS, TC_BLK), lambda i: (0, 0)),
            pl.BlockSpec((TC_BLK, D), lambda i: (i, 0)),
            pl.BlockSpec((D, 1), lambda i: (0, 0)),
            pl.BlockSpec((1,), lambda i: (0,)),
        ],
        out_specs=pl.BlockSpec((NUM_SEGMENTS, D), lambda i: (0, 0)),
        out_shape=jax.ShapeDtypeStruct((NUM_SEGMENTS, D), jnp.float32),
    )(bb, iota_b, x, W.astype(jnp.float32), b.astype(jnp.float32))


def _merge_body(p_ref, q_ref, o_ref):
    o_ref[...] = p_ref[0] + p_ref[1] + q_ref[...]


def kernel(x, batch, W, b):
    n_rows, d = x.shape
    assert d == D and n_rows % 8 == 0 and TC_SPLIT % 8 == 0
    nw = 32
    n_oct = (n_rows - TC_SPLIT) // 8
    max_rows = 8 * max((w + 1) * n_oct // nw - w * n_oct // nw for w in range(nw))
    n_chunks = -(-max_rows // CHUNK)
    wb = jnp.concatenate(
        [
            W.reshape(D).astype(jnp.float32),
            jnp.broadcast_to(b.reshape(-1)[:1].astype(jnp.float32), (LANES,)),
        ]
    )
    sc_partials = _sc_weighted_segment_sum(
        x, batch.astype(jnp.int32), wb,
        row0=TC_SPLIT, n_rows=n_rows, n_chunks=n_chunks,
    )
    tc_partial = _tc_weighted_segment_sum(x, batch, W, b)
    return pl.pallas_call(
        _merge_body,
        out_shape=jax.ShapeDtypeStruct((NUM_SEGMENTS, D), jnp.float32),
    )(sc_partials, tc_partial)


# speedup vs baseline: 1.1377x; 1.1377x over previous
import functools

import jax
import jax.numpy as jnp
from jax import lax
from jax.experimental import pallas as pl
from jax.experimental.pallas import tpu as pltpu
from jax.experimental.pallas import tpu_sc as plsc

NUM_SEGMENTS = 256
D = 128
LANES = 16
NVR = D // LANES
CHUNK = 256
ACC_ROWS = NUM_SEGMENTS + LANES


def _sc_weighted_segment_sum(x, batch, wb, *, row0, n_rows, n_chunks):
    mesh = plsc.VectorSubcoreMesh(core_axis_name="c", subcore_axis_name="s")
    info = plsc.get_sparse_core_info()
    nc, ns = info.num_cores, info.num_subcores
    nw = nc * ns
    oct0 = row0 // 8
    n_oct = (n_rows - row0) // 8

    @functools.partial(
        pl.kernel,
        mesh=mesh,
        out_type=jax.ShapeDtypeStruct((nc, NUM_SEGMENTS, D), jnp.float32),
        scratch_types=[
            pltpu.VMEM((CHUNK, D), jnp.float32),
            pltpu.VMEM((CHUNK, D), jnp.float32),
            pltpu.VMEM((CHUNK + 8,), jnp.int32),
            pltpu.VMEM((CHUNK + 8,), jnp.int32),
            pltpu.VMEM((D + LANES,), jnp.float32),
            pltpu.VMEM((ACC_ROWS, D), jnp.float32),
            pltpu.VMEM((ns, D), jnp.float32),
            pltpu.VMEM((LANES,), jnp.int32),
            pltpu.VMEM_SHARED((NUM_SEGMENTS + 1, D), jnp.float32),
            pltpu.SemaphoreType.DMA,
            pltpu.SemaphoreType.DMA,
            pltpu.SemaphoreType.DMA,
            pltpu.SemaphoreType.DMA,
        ],
    )
    def run(x_hbm, b_hbm, wb_hbm, out_hbm,
            xb0, xb1, bb0, bb1, wv, accum, zbuf, sbi, shared,
            sx0, sx1, sb0, sb1):
        cid = lax.axis_index("c")
        sid = lax.axis_index("s")
        wid = sid * nc + cid

        xbufs = (xb0, xb1)
        bbufs = (bb0, bb1)
        sxs = (sx0, sx1)
        sbs = (sb0, sb1)

        o0 = oct0 + wid * n_oct // nw
        o1 = oct0 + (wid + 1) * n_oct // nw
        r0 = o0 * 8
        r1 = o1 * 8

        pltpu.sync_copy(wb_hbm, wv)

        zvec = jnp.zeros((LANES,), jnp.float32)

        def zacc_body(r, _):
            for d in range(NVR):
                accum[r, pl.ds(d * LANES, LANES)] = zvec
            return 0
        lax.fori_loop(0, ACC_ROWS, zacc_body, 0)

        for r in range(ns):
            for d in range(NVR):
                zbuf[r, pl.ds(d * LANES, LANES)] = zvec
        pltpu.sync_copy(zbuf, shared.at[pl.ds(sid * ns, ns)])
        plsc.subcore_barrier()

        def chunk_base(k):
            return jnp.minimum(r0 + k * CHUNK, r1 - CHUNK)

        def start_copy(k, parity):
            base = chunk_base(k)
            pltpu.make_async_copy(
                x_hbm.at[pl.ds(base, CHUNK)], xbufs[parity], sxs[parity]
            ).start()
            pltpu.make_async_copy(
                b_hbm.at[pl.ds(base, CHUNK)],
                bbufs[parity].at[pl.ds(0, CHUNK)],
                sbs[parity],
            ).start()

        def wait_copy(k, parity):
            base = chunk_base(k)
            pltpu.make_async_copy(
                x_hbm.at[pl.ds(base, CHUNK)], xbufs[parity], sxs[parity]
            ).wait()
            pltpu.make_async_copy(
                b_hbm.at[pl.ds(base, CHUNK)],
                bbufs[parity].at[pl.ds(0, CHUNK)],
                sbs[parity],
            ).wait()

        wregs = [wv[pl.ds(d * LANES, LANES)] for d in range(NVR)]
        bias = wv[pl.ds(D, LANES)]

        lane_iota = lax.iota(jnp.int32, LANES)
        perms = [lane_iota ^ sh for sh in (8, 4, 2, 1)]

        def lane_sum(m):
            for p in perms:
                m = m + m.at[p].get(mode="promise_in_bounds")
            return m

        def flush(cur_seg, acc):
            @pl.when(cur_seg >= 0)
            def _():
                for d in range(NVR):
                    sl = pl.ds(d * LANES, LANES)
                    accum[cur_seg, sl] = accum[cur_seg, sl] + acc[d]

        def process_chunk(carry, parity, base, fresh, tail):
            xb = xbufs[parity]
            bb = bbufs[parity]

            def grp_body(g, c):
                segv = bb[pl.ds(g * 16, LANES)]
                for i in range(16):
                    cur_seg = c[0]
                    acc = c[1:]
                    j = g * 16 + i
                    xv = [xb[j, pl.ds(d * LANES, LANES)] for d in range(NVR)]
                    m = xv[0] * wregs[0]
                    for d in range(1, NVR):
                        m = m + xv[d] * wregs[d]
                    z = lane_sum(m) + bias
                    w = 1.0 / (1.0 + jnp.exp(-z))
                    if tail:
                        valid = ((base + j) >= fresh).astype(jnp.float32)
                        w = w * valid
                    contrib = [w * xv[d] for d in range(NVR)]
                    seg = segv[i]
                    is_new = seg != cur_seg

                    @pl.when(jnp.logical_and(is_new, cur_seg >= 0))
                    def _(cur_seg=cur_seg, acc=acc):
                        for d in range(NVR):
                            sl = pl.ds(d * LANES, LANES)
                            accum[cur_seg, sl] = accum[cur_seg, sl] + acc[d]

                    c = (seg,) + tuple(
                        jnp.where(is_new, cb, a + cb)
                        for a, cb in zip(acc, contrib)
                    )
                return c

            return lax.fori_loop(0, CHUNK // 16, grp_body, carry)

        pltpu.sync_copy(b_hbm.at[pl.ds(r0, LANES)], sbi)
        seg_lo = sbi[pl.ds(0, LANES)][0]

        start_copy(0, 0)
        carry = (jnp.int32(-1),) + tuple(
            jnp.zeros((LANES,), jnp.float32) for _ in range(NVR)
        )

        n_pairs = (n_chunks - 1) // 2

        def pair_body(k2, c):
            for p in range(2):
                k = 2 * k2 + p
                wait_copy(k, p)
                start_copy(k + 1, (p + 1) % 2)
                c = process_chunk(c, p, chunk_base(k), r0 + k * CHUNK,
                                  tail=False)
            return c

        carry = lax.fori_loop(0, n_pairs, pair_body, carry)
        for k in range(2 * n_pairs, n_chunks):
            wait_copy(k, k % 2)
            if k + 1 < n_chunks:
                start_copy(k + 1, (k + 1) % 2)
            carry = process_chunk(carry, k % 2, chunk_base(k),
                                  r0 + k * CHUNK, tail=k == n_chunks - 1)

        seg_hi = carry[0]
        flush(seg_hi, carry[1:])

        iota = lax.iota(jnp.int32, LANES)
        for w in range(ns):
            start = seg_lo + w * LANES

            @pl.when(start <= seg_hi)
            def _(start=start):
                idxvec = jnp.minimum(start + iota, NUM_SEGMENTS)
                pltpu.sync_copy(
                    accum.at[pl.ds(start, LANES)],
                    shared.at[idxvec],
                    add=True,
                )

        plsc.subcore_barrier()

        pltpu.sync_copy(shared.at[pl.ds(sid * ns, ns)], zbuf)
        pltpu.sync_copy(zbuf, out_hbm.at[cid].at[pl.ds(sid * ns, ns)])

    return run(x, batch, wb)


TC_BLK = 2048
TC_SPLIT = 61440


def _tc_body(bb_ref, x_ref, w_ref, b_ref, o_ref, io_ref):
    i = pl.program_id(0)

    @pl.when(i == 0)
    def _():
        io_ref[...] = lax.broadcasted_iota(
            jnp.int32, (NUM_SEGMENTS, TC_BLK), 0
        ).astype(jnp.bfloat16)
        o_ref[...] = jnp.zeros_like(o_ref)

    xb = x_ref[...]
    z = jnp.dot(xb, w_ref[...], preferred_element_type=jnp.float32)
    z = z + b_ref[...]
    wgt = 1.0 / (1.0 + jnp.exp(-z))
    y = (wgt * xb).astype(jnp.bfloat16)
    segb = bb_ref[...].astype(jnp.bfloat16)
    onehot = (io_ref[...] == segb[None, :]).astype(jnp.bfloat16)
    part = jnp.dot(onehot, y, preferred_element_type=jnp.float32)
    o_ref[...] += part


def _tc_weighted_segment_sum(x, batch_i32, W, b):
    grid = TC_SPLIT // TC_BLK
    return pl.pallas_call(
        _tc_body,
        grid=(grid,),
        in_specs=[
            pl.BlockSpec((TC_BLK,), lambda i: (i,)),
            pl.BlockSpec((TC_BLK, D), lambda i: (i, 0)),
            pl.BlockSpec((D, 1), lambda i: (0, 0)),
            pl.BlockSpec((1,), lambda i: (0,)),
        ],
        out_specs=pl.BlockSpec((NUM_SEGMENTS, D), lambda i: (0, 0)),
        out_shape=jax.ShapeDtypeStruct((NUM_SEGMENTS, D), jnp.float32),
        scratch_shapes=[pltpu.VMEM((NUM_SEGMENTS, TC_BLK), jnp.bfloat16)],
    )(batch_i32, x, W.astype(jnp.float32), b.astype(jnp.float32))


def _merge_body(p_ref, q_ref, o_ref):
    o_ref[...] = p_ref[0] + p_ref[1] + q_ref[...]


def kernel(x, batch, W, b):
    n_rows, d = x.shape
    assert d == D and n_rows % 8 == 0 and TC_SPLIT % 8 == 0
    nw = 32
    n_oct = (n_rows - TC_SPLIT) // 8
    max_rows = 8 * max((w + 1) * n_oct // nw - w * n_oct // nw for w in range(nw))
    n_chunks = -(-max_rows // CHUNK)
    wb = jnp.concatenate(
        [
            W.reshape(D).astype(jnp.float32),
            jnp.broadcast_to(b.reshape(-1)[:1].astype(jnp.float32), (LANES,)),
        ]
    )
    batch_i32 = batch.astype(jnp.int32)
    sc_partials = _sc_weighted_segment_sum(
        x, batch_i32, wb,
        row0=TC_SPLIT, n_rows=n_rows, n_chunks=n_chunks,
    )
    tc_partial = _tc_weighted_segment_sum(x, batch_i32, W, b)
    return pl.pallas_call(
        _merge_body,
        out_shape=jax.ShapeDtypeStruct((NUM_SEGMENTS, D), jnp.float32),
    )(sc_partials, tc_partial)


# speedup vs baseline: 1.2343x; 1.0849x over previous
import functools

import jax
import jax.numpy as jnp
from jax import lax
from jax.experimental import pallas as pl
from jax.experimental.pallas import tpu as pltpu
from jax.experimental.pallas import tpu_sc as plsc

NUM_SEGMENTS = 256
D = 128
LANES = 16
NVR = D // LANES
CHUNK = 256
ACC_ROWS = NUM_SEGMENTS + LANES


def _sc_weighted_segment_sum(x, batch, wb, *, row0, n_rows, n_chunks):
    mesh = plsc.VectorSubcoreMesh(core_axis_name="c", subcore_axis_name="s")
    info = plsc.get_sparse_core_info()
    nc, ns = info.num_cores, info.num_subcores
    nw = nc * ns
    oct0 = row0 // 8
    n_oct = (n_rows - row0) // 8

    @functools.partial(
        pl.kernel,
        mesh=mesh,
        out_type=jax.ShapeDtypeStruct((nc, NUM_SEGMENTS, D), jnp.float32),
        scratch_types=[
            pltpu.VMEM((CHUNK, D), jnp.float32),
            pltpu.VMEM((CHUNK, D), jnp.float32),
            pltpu.VMEM((CHUNK + 8,), jnp.int32),
            pltpu.VMEM((CHUNK + 8,), jnp.int32),
            pltpu.VMEM((D + LANES,), jnp.float32),
            pltpu.VMEM((ACC_ROWS, D), jnp.float32),
            pltpu.VMEM((ns, D), jnp.float32),
            pltpu.VMEM((LANES,), jnp.int32),
            pltpu.VMEM_SHARED((NUM_SEGMENTS + 1, D), jnp.float32),
            pltpu.SemaphoreType.DMA,
            pltpu.SemaphoreType.DMA,
            pltpu.SemaphoreType.DMA,
            pltpu.SemaphoreType.DMA,
        ],
    )
    def run(x_hbm, b_hbm, wb_hbm, out_hbm,
            xb0, xb1, bb0, bb1, wv, accum, zbuf, sbi, shared,
            sx0, sx1, sb0, sb1):
        cid = lax.axis_index("c")
        sid = lax.axis_index("s")
        wid = sid * nc + cid

        xbufs = (xb0, xb1)
        bbufs = (bb0, bb1)
        sxs = (sx0, sx1)
        sbs = (sb0, sb1)

        o0 = oct0 + wid * n_oct // nw
        o1 = oct0 + (wid + 1) * n_oct // nw
        r0 = o0 * 8
        r1 = o1 * 8

        pltpu.sync_copy(wb_hbm, wv)

        zvec = jnp.zeros((LANES,), jnp.float32)

        def zacc_body(r, _):
            for d in range(NVR):
                accum[r, pl.ds(d * LANES, LANES)] = zvec
            return 0
        lax.fori_loop(0, ACC_ROWS, zacc_body, 0)

        for r in range(ns):
            for d in range(NVR):
                zbuf[r, pl.ds(d * LANES, LANES)] = zvec
        pltpu.sync_copy(zbuf, shared.at[pl.ds(sid * ns, ns)])
        plsc.subcore_barrier()

        def chunk_base(k):
            return jnp.minimum(r0 + k * CHUNK, r1 - CHUNK)

        def start_copy(k, parity):
            base = chunk_base(k)
            pltpu.make_async_copy(
                x_hbm.at[pl.ds(base, CHUNK)], xbufs[parity], sxs[parity]
            ).start()
            pltpu.make_async_copy(
                b_hbm.at[pl.ds(base, CHUNK)],
                bbufs[parity].at[pl.ds(0, CHUNK)],
                sbs[parity],
            ).start()

        def wait_copy(k, parity):
            base = chunk_base(k)
            pltpu.make_async_copy(
                x_hbm.at[pl.ds(base, CHUNK)], xbufs[parity], sxs[parity]
            ).wait()
            pltpu.make_async_copy(
                b_hbm.at[pl.ds(base, CHUNK)],
                bbufs[parity].at[pl.ds(0, CHUNK)],
                sbs[parity],
            ).wait()

        wregs = [wv[pl.ds(d * LANES, LANES)] for d in range(NVR)]
        bias = wv[pl.ds(D, LANES)]

        lane_iota = lax.iota(jnp.int32, LANES)
        perms = [lane_iota ^ sh for sh in (8, 4, 2, 1)]

        def lane_sum(m):
            for p in perms:
                m = m + m.at[p].get(mode="promise_in_bounds")
            return m

        def flush(cur_seg, acc):
            @pl.when(cur_seg >= 0)
            def _():
                for d in range(NVR):
                    sl = pl.ds(d * LANES, LANES)
                    accum[cur_seg, sl] = accum[cur_seg, sl] + acc[d]

        def process_chunk(carry, parity, base, fresh, tail):
            xb = xbufs[parity]
            bb = bbufs[parity]

            def grp_body(g, c):
                segv = bb[pl.ds(g * 16, LANES)]
                for i in range(16):
                    cur_seg = c[0]
                    acc = c[1:]
                    j = g * 16 + i
                    xv = [xb[j, pl.ds(d * LANES, LANES)] for d in range(NVR)]
                    m = xv[0] * wregs[0]
                    for d in range(1, NVR):
                        m = m + xv[d] * wregs[d]
                    z = lane_sum(m) + bias
                    w = 1.0 / (1.0 + jnp.exp(-z))
                    if tail:
                        valid = ((base + j) >= fresh).astype(jnp.float32)
                        w = w * valid
                    contrib = [w * xv[d] for d in range(NVR)]
                    seg = segv[i]
                    is_new = seg != cur_seg

                    @pl.when(jnp.logical_and(is_new, cur_seg >= 0))
                    def _(cur_seg=cur_seg, acc=acc):
                        for d in range(NVR):
                            sl = pl.ds(d * LANES, LANES)
                            accum[cur_seg, sl] = accum[cur_seg, sl] + acc[d]

                    c = (seg,) + tuple(
                        jnp.where(is_new, cb, a + cb)
                        for a, cb in zip(acc, contrib)
                    )
                return c

            return lax.fori_loop(0, CHUNK // 16, grp_body, carry)

        pltpu.sync_copy(b_hbm.at[pl.ds(r0, LANES)], sbi)
        seg_lo = sbi[pl.ds(0, LANES)][0]

        start_copy(0, 0)
        carry = (jnp.int32(-1),) + tuple(
            jnp.zeros((LANES,), jnp.float32) for _ in range(NVR)
        )

        n_pairs = (n_chunks - 1) // 2

        def pair_body(k2, c):
            for p in range(2):
                k = 2 * k2 + p
                wait_copy(k, p)
                start_copy(k + 1, (p + 1) % 2)
                c = process_chunk(c, p, chunk_base(k), r0 + k * CHUNK,
                                  tail=False)
            return c

        carry = lax.fori_loop(0, n_pairs, pair_body, carry)
        for k in range(2 * n_pairs, n_chunks):
            wait_copy(k, k % 2)
            if k + 1 < n_chunks:
                start_copy(k + 1, (k + 1) % 2)
            carry = process_chunk(carry, k % 2, chunk_base(k),
                                  r0 + k * CHUNK, tail=k == n_chunks - 1)

        seg_hi = carry[0]
        flush(seg_hi, carry[1:])

        iota = lax.iota(jnp.int32, LANES)
        for w in range(ns):
            start = seg_lo + w * LANES

            @pl.when(start <= seg_hi)
            def _(start=start):
                idxvec = jnp.minimum(start + iota, NUM_SEGMENTS)
                pltpu.sync_copy(
                    accum.at[pl.ds(start, LANES)],
                    shared.at[idxvec],
                    add=True,
                )

        plsc.subcore_barrier()

        pltpu.sync_copy(shared.at[pl.ds(sid * ns, ns)], zbuf)
        pltpu.sync_copy(zbuf, out_hbm.at[cid].at[pl.ds(sid * ns, ns)])

    return run(x, batch, wb)


TC_BLK = 2048
TC_SPLIT = 67584


def _tc_body(bb_ref, x_ref, w_ref, b_ref, o_ref, io_ref):
    i = pl.program_id(0)

    @pl.when(i == 0)
    def _():
        io_ref[...] = lax.broadcasted_iota(
            jnp.int32, (NUM_SEGMENTS, TC_BLK), 0
        ).astype(jnp.bfloat16)
        o_ref[...] = jnp.zeros_like(o_ref)

    xb = x_ref[...]
    z = jnp.dot(xb, w_ref[...], preferred_element_type=jnp.float32)
    z = z + b_ref[...]
    wgt = 1.0 / (1.0 + jnp.exp(-z))
    y = (wgt * xb).astype(jnp.bfloat16)
    segb = bb_ref[...].astype(jnp.bfloat16)
    onehot = (io_ref[...] == segb[None, :]).astype(jnp.bfloat16)
    part = jnp.dot(onehot, y, preferred_element_type=jnp.float32)
    o_ref[...] += part


def _tc_weighted_segment_sum(x, batch_i32, W, b):
    grid = TC_SPLIT // TC_BLK
    return pl.pallas_call(
        _tc_body,
        grid=(grid,),
        in_specs=[
            pl.BlockSpec((TC_BLK,), lambda i: (i,)),
            pl.BlockSpec((TC_BLK, D), lambda i: (i, 0)),
            pl.BlockSpec((D, 1), lambda i: (0, 0)),
            pl.BlockSpec((1,), lambda i: (0,)),
        ],
        out_specs=pl.BlockSpec((NUM_SEGMENTS, D), lambda i: (0, 0)),
        out_shape=jax.ShapeDtypeStruct((NUM_SEGMENTS, D), jnp.float32),
        scratch_shapes=[pltpu.VMEM((NUM_SEGMENTS, TC_BLK), jnp.bfloat16)],
    )(batch_i32, x, W.astype(jnp.float32), b.astype(jnp.float32))


def _merge_body(p_ref, q_ref, o_ref):
    o_ref[...] = p_ref[0] + p_ref[1] + q_ref[...]


def kernel(x, batch, W, b):
    n_rows, d = x.shape
    assert d == D and n_rows % 8 == 0 and TC_SPLIT % 8 == 0
    nw = 32
    n_oct = (n_rows - TC_SPLIT) // 8
    max_rows = 8 * max((w + 1) * n_oct // nw - w * n_oct // nw for w in range(nw))
    n_chunks = -(-max_rows // CHUNK)
    wb = jnp.concatenate(
        [
            W.reshape(D).astype(jnp.float32),
            jnp.broadcast_to(b.reshape(-1)[:1].astype(jnp.float32), (LANES,)),
        ]
    )
    batch_i32 = batch.astype(jnp.int32)
    sc_partials = _sc_weighted_segment_sum(
        x, batch_i32, wb,
        row0=TC_SPLIT, n_rows=n_rows, n_chunks=n_chunks,
    )
    tc_partial = _tc_weighted_segment_sum(x, batch_i32, W, b)
    return pl.pallas_call(
        _merge_body,
        out_shape=jax.ShapeDtypeStruct((NUM_SEGMENTS, D), jnp.float32),
    )(sc_partials, tc_partial)
